# baseline (speedup 1.0000x reference)
import functools

import jax
import jax.numpy as jnp
import numpy as np
from jax import lax
from jax.experimental import pallas as pl
from jax.experimental.pallas import tpu as pltpu
from jax.experimental.pallas import tpu_sc as plsc

_NC, _NS, _L = 2, 16, 16
_NW = _NC * _NS
_N = 200000
_CH = 200
_NCHUNK = _N // _CH


_GDN = lax.GatherDimensionNumbers(
    offset_dims=(), collapsed_slice_dims=(0,), start_index_map=(0,)
)


def _vtake(vec, idx):
    return lax.gather(
        vec,
        idx[:, None],
        dimension_numbers=_GDN,
        slice_sizes=(1,),
        mode=lax.GatherScatterMode.PROMISE_IN_BOUNDS,
    )


def _sc_body(in0_hbm, in1_hbm, out_hbm, in0_v, in1_v, out_v, sem):
    wid = lax.axis_index("s") * _NC + lax.axis_index("c")

    lane = lax.iota(jnp.int32, 16)
    half = lane >> 3
    idx_w = [2 * k + half for k in range(8)]
    rot8 = lane ^ 8

    def row_body(r, carry):
        a = in0_v[r, pl.ds(0, 16)] * 0.5
        b = in0_v[r, pl.ds(16, 16)] * 0.75
        acc0 = None
        acc1 = None
        for k in range(8):
            ra = in1_v[r, pl.ds(16 * k, 16)]
            rb = in1_v[r, pl.ds(128 + 16 * k, 16)]
            m0 = ra + 0.5 * rb
            m1 = ra - (1.0 / 3.0) * rb
            p0 = _vtake(a, idx_w[k]) * m0
            p1 = _vtake(b, idx_w[k]) * m1
            acc0 = p0 if acc0 is None else acc0 + p0
            acc1 = p1 if acc1 is None else acc1 + p1
        f0 = acc0 + _vtake(acc0, rot8)
        f1 = acc1 + _vtake(acc1, rot8)
        out_v[r, :] = jnp.where(lane < 8, f0, f1)
        return carry

    def chunk_body(i, carry):
        r0 = (wid + i * _NW) * _CH
        pltpu.async_copy(in0_hbm.at[pl.ds(r0, _CH)], in0_v, sem).wait()
        pltpu.async_copy(in1_hbm.at[pl.ds(r0, _CH)], in1_v, sem).wait()
        lax.fori_loop(0, _CH, row_body, 0)
        pltpu.async_copy(out_v, out_hbm.at[pl.ds(r0, _CH)], sem).wait()
        return carry

    n_mine = (_NCHUNK - wid + _NW - 1) // _NW
    lax.fori_loop(0, n_mine, chunk_body, 0)


@jax.jit
def kernel(in0, in1):
    n = in0.shape[0]
    f = pl.kernel(
        _sc_body,
        out_type=jax.ShapeDtypeStruct((n, 16), jnp.float32),
        mesh=plsc.VectorSubcoreMesh(core_axis_name="c", subcore_axis_name="s"),
        scratch_types=[
            pltpu.VMEM((_CH, 32), jnp.float32),
            pltpu.VMEM((_CH, 256), jnp.float32),
            pltpu.VMEM((_CH, 16), jnp.float32),
            pltpu.SemaphoreType.DMA,
        ],
        compiler_params=pltpu.CompilerParams(use_tc_tiling_on_sc=True),
    )
    return f(in0, in1)


# speedup vs baseline: 1.6331x; 1.6331x over previous
import functools

import jax
import jax.numpy as jnp
from jax import lax
from jax.experimental import pallas as pl
from jax.experimental.pallas import tpu as pltpu
from jax.experimental.pallas import tpu_sc as plsc

_NC, _NS = 2, 16
_NW = _NC * _NS
_N = 200000
_CH = 80
_NCHUNK = _N // _CH

_GDN = lax.GatherDimensionNumbers(
    offset_dims=(), collapsed_slice_dims=(0,), start_index_map=(0,)
)


def _vtake(vec, idx):
    return lax.gather(
        vec,
        idx[:, None],
        dimension_numbers=_GDN,
        slice_sizes=(1,),
        mode=lax.GatherScatterMode.PROMISE_IN_BOUNDS,
    )


def _sc_body(
    in0_hbm,
    in1_hbm,
    out_hbm,
    in0_v,
    in1_v,
    out_v,
    sem_i0a,
    sem_i0b,
    sem_i1a,
    sem_i1b,
    sem_oa,
    sem_ob,
):
    wid = lax.axis_index("s") * _NC + lax.axis_index("c")
    n_mine = (_NCHUNK - wid + _NW - 1) // _NW

    sem_i0 = (sem_i0a, sem_i0b)
    sem_i1 = (sem_i1a, sem_i1b)
    sem_o = (sem_oa, sem_ob)

    lane = lax.iota(jnp.int32, 16)
    half = lane >> 3
    idx_w = [2 * k + half for k in range(8)]
    rot8 = lane ^ 8

    def chunk_r0(i):
        return (wid + i * _NW) * _CH

    def start_in(i, b):
        r0 = chunk_r0(i)
        pltpu.make_async_copy(
            in0_hbm.at[pl.ds(r0, _CH)], in0_v.at[b], sem_i0[b]
        ).start()
        pltpu.make_async_copy(
            in1_hbm.at[pl.ds(r0, _CH)], in1_v.at[b], sem_i1[b]
        ).start()

    def wait_in(i, b):
        r0 = chunk_r0(i)
        pltpu.make_async_copy(
            in0_hbm.at[pl.ds(r0, _CH)], in0_v.at[b], sem_i0[b]
        ).wait()
        pltpu.make_async_copy(
            in1_hbm.at[pl.ds(r0, _CH)], in1_v.at[b], sem_i1[b]
        ).wait()

    def start_out(i, b):
        r0 = chunk_r0(i)
        pltpu.make_async_copy(
            out_v.at[b], out_hbm.at[pl.ds(r0, _CH)], sem_o[b]
        ).start()

    def wait_out(i, b):
        r0 = chunk_r0(i)
        pltpu.make_async_copy(
            out_v.at[b], out_hbm.at[pl.ds(r0, _CH)], sem_o[b]
        ).wait()

    def compute_chunk(b):
        @plsc.parallel_loop(0, _CH, unroll=4)
        def _row(r):
            a = in0_v[b, r, pl.ds(0, 16)] * 0.5
            bb = in0_v[b, r, pl.ds(16, 16)] * 0.75
            acc0 = None
            acc1 = None
            for k in range(8):
                ra = in1_v[b, r, pl.ds(16 * k, 16)]
                rb = in1_v[b, r, pl.ds(128 + 16 * k, 16)]
                m0 = ra + 0.5 * rb
                m1 = ra - (1.0 / 3.0) * rb
                p0 = _vtake(a, idx_w[k]) * m0
                p1 = _vtake(bb, idx_w[k]) * m1
                acc0 = p0 if acc0 is None else acc0 + p0
                acc1 = p1 if acc1 is None else acc1 + p1
            f0 = acc0 + _vtake(acc0, rot8)
            f1 = acc1 + _vtake(acc1, rot8)
            out_v[b, r, :] = jnp.where(lane < 8, f0, f1)

    start_in(0, 0)

    def pair_body(p, carry):
        for b in range(2):
            i = 2 * p + b

            @pl.when(i < n_mine)
            def _():
                wait_in(i, b)

                @pl.when(i + 1 < n_mine)
                def _():
                    start_in(i + 1, 1 - b)

                @pl.when(i >= 2)
                def _():
                    wait_out(i - 2, b)

                compute_chunk(b)
                start_out(i, b)

        return carry

    lax.fori_loop(0, (n_mine + 1) // 2, pair_body, 0)
    wait_out(0, 0)
    wait_out(0, 1)


@jax.jit
def kernel(in0, in1):
    n = in0.shape[0]
    f = pl.kernel(
        _sc_body,
        out_type=jax.ShapeDtypeStruct((n, 16), jnp.float32),
        mesh=plsc.VectorSubcoreMesh(core_axis_name="c", subcore_axis_name="s"),
        scratch_types=[
            pltpu.VMEM((2, _CH, 32), jnp.float32),
            pltpu.VMEM((2, _CH, 256), jnp.float32),
            pltpu.VMEM((2, _CH, 16), jnp.float32),
            pltpu.SemaphoreType.DMA,
            pltpu.SemaphoreType.DMA,
            pltpu.SemaphoreType.DMA,
            pltpu.SemaphoreType.DMA,
            pltpu.SemaphoreType.DMA,
            pltpu.SemaphoreType.DMA,
        ],
        compiler_params=pltpu.CompilerParams(use_tc_tiling_on_sc=True),
    )
    return f(in0, in1)


# speedup vs baseline: 1.6938x; 1.0372x over previous
import functools

import jax
import jax.numpy as jnp
from jax.experimental import pallas as pl
from jax.experimental.pallas import tpu as pltpu

_C = ((0.5, 0.25), (0.75, -0.25))
_BLOCK_ROWS = 2000


def _body(in0_ref, in1_ref, out_ref):
    in0 = in0_ref[...]
    in1 = in1_ref[...]

    k_t = jax.lax.broadcasted_iota(jnp.int32, (32, 256), 0)
    l_t = jax.lax.broadcasted_iota(jnp.int32, (32, 256), 1)
    same_t = (l_t // 128) == (k_t // 16)
    same_u = ((l_t % 128) // 8) == (k_t % 16)
    scale = jnp.where(k_t // 16 == 0, _C[0][0], _C[1][0]).astype(jnp.float32)
    B = jnp.where(same_t & same_u, scale, 0.0)

    r_i = jax.lax.broadcasted_iota(jnp.int32, (256, 16), 0)
    c_i = jax.lax.broadcasted_iota(jnp.int32, (256, 16), 1)
    S = jnp.where(
        ((r_i // 128) == (c_i // 8)) & ((r_i % 8) == (c_i % 8)), 1.0, 0.0
    ).astype(jnp.float32)

    in1a = in1[:, :128]
    in1b = in1[:, 128:]
    m0 = in1a + (_C[0][1] / _C[0][0]) * in1b
    m1 = in1a + (_C[1][1] / _C[1][0]) * in1b
    m = jnp.concatenate([m0, m1], axis=1)

    w = jax.lax.dot(in0, B, precision=jax.lax.Precision.DEFAULT)
    out_ref[...] = jax.lax.dot(
        w * m, S, precision=jax.lax.Precision.DEFAULT
    )


@jax.jit
def kernel(in0, in1):
    n = in0.shape[0]
    r = _BLOCK_ROWS
    grid = (pl.cdiv(n, r),)
    return pl.pallas_call(
        _body,
        grid=grid,
        in_specs=[
            pl.BlockSpec((r, 32), lambda i: (i, 0)),
            pl.BlockSpec((r, 256), lambda i: (i, 0)),
        ],
        out_specs=pl.BlockSpec((r, 16), lambda i: (i, 0)),
        out_shape=jax.ShapeDtypeStruct((n, 16), in0.dtype),
        compiler_params=pltpu.CompilerParams(
            dimension_semantics=("arbitrary",),
        ),
    )(in0, in1)
